# baseline (speedup 1.0000x reference)
import jax
import jax.numpy as jnp
from jax.experimental import pallas as pl
from jax.experimental.pallas import tpu as pltpu

N = 10000
D = 128
BM = 400


def _gcn_body(adj_ref, emb_ref, out_ref):
    a = adj_ref[...].astype(jnp.bfloat16)
    out_ref[...] = jnp.dot(a, emb_ref[...], preferred_element_type=jnp.float32)


def kernel(adj, embeds):
    emb_bf16 = embeds.astype(jnp.bfloat16)
    grid = (N // BM,)
    return pl.pallas_call(
        _gcn_body,
        grid=grid,
        in_specs=[
            pl.BlockSpec((BM, N), lambda i: (i, 0)),
            pl.BlockSpec((N, D), lambda i: (0, 0)),
        ],
        out_specs=pl.BlockSpec((BM, D), lambda i: (i, 0)),
        out_shape=jax.ShapeDtypeStruct((N, D), jnp.float32),
        compiler_params=pltpu.CompilerParams(
            dimension_semantics=("arbitrary",),
        ),
    )(adj, emb_bf16)


# speedup vs baseline: 1.0053x; 1.0053x over previous
import jax
import jax.numpy as jnp
from jax.experimental import pallas as pl
from jax.experimental.pallas import tpu as pltpu

N = 10000
D = 128
BM = 200


def _gcn_body(adj_ref, emb_ref, out_ref):
    a = adj_ref[...].astype(jnp.bfloat16)
    out_ref[...] = jnp.dot(a, emb_ref[...], preferred_element_type=jnp.float32)


def kernel(adj, embeds):
    emb_bf16 = embeds.astype(jnp.bfloat16)
    grid = (N // BM,)
    return pl.pallas_call(
        _gcn_body,
        grid=grid,
        in_specs=[
            pl.BlockSpec((BM, N), lambda i: (i, 0)),
            pl.BlockSpec((N, D), lambda i: (0, 0)),
        ],
        out_specs=pl.BlockSpec((BM, D), lambda i: (i, 0)),
        out_shape=jax.ShapeDtypeStruct((N, D), jnp.float32),
        compiler_params=pltpu.CompilerParams(
            dimension_semantics=("arbitrary",),
        ),
    )(adj, emb_bf16)


# speedup vs baseline: 1.0231x; 1.0177x over previous
import jax
import jax.numpy as jnp
from jax.experimental import pallas as pl
from jax.experimental.pallas import tpu as pltpu

N = 10000
D = 128
BM = 400


def _gcn_body(adj_ref, emb_ref, out_ref, emb_bf_ref):
    @pl.when(pl.program_id(0) == 0)
    def _():
        emb_bf_ref[...] = emb_ref[...].astype(jnp.bfloat16)

    a = adj_ref[...].astype(jnp.bfloat16)
    out_ref[...] = jnp.dot(a, emb_bf_ref[...], preferred_element_type=jnp.float32)


def kernel(adj, embeds):
    grid = (N // BM,)
    return pl.pallas_call(
        _gcn_body,
        grid=grid,
        in_specs=[
            pl.BlockSpec((BM, N), lambda i: (i, 0)),
            pl.BlockSpec((N, D), lambda i: (0, 0)),
        ],
        out_specs=pl.BlockSpec((BM, D), lambda i: (i, 0)),
        out_shape=jax.ShapeDtypeStruct((N, D), jnp.float32),
        scratch_shapes=[pltpu.VMEM((N, D), jnp.bfloat16)],
        compiler_params=pltpu.CompilerParams(
            dimension_semantics=("arbitrary",),
        ),
    )(adj, embeds)


# speedup vs baseline: 1.0291x; 1.0059x over previous
import jax
import jax.numpy as jnp
from jax.experimental import pallas as pl
from jax.experimental.pallas import tpu as pltpu

N = 10000
D = 128
BM = 200


def _gcn_body(adj_ref, emb_ref, out_ref, emb_bf_ref):
    @pl.when(pl.program_id(0) == 0)
    def _():
        emb_bf_ref[...] = emb_ref[...].astype(jnp.bfloat16)

    a = adj_ref[...].astype(jnp.bfloat16)
    out_ref[...] = jnp.dot(a, emb_bf_ref[...], preferred_element_type=jnp.float32)


def kernel(adj, embeds):
    grid = (N // BM,)
    return pl.pallas_call(
        _gcn_body,
        grid=grid,
        in_specs=[
            pl.BlockSpec((BM, N), lambda i: (i, 0)),
            pl.BlockSpec((N, D), lambda i: (0, 0)),
        ],
        out_specs=pl.BlockSpec((BM, D), lambda i: (i, 0)),
        out_shape=jax.ShapeDtypeStruct((N, D), jnp.float32),
        scratch_shapes=[pltpu.VMEM((N, D), jnp.bfloat16)],
        compiler_params=pltpu.CompilerParams(
            dimension_semantics=("arbitrary",),
        ),
    )(adj, embeds)
